# baseline (speedup 1.0000x reference)
import functools

import jax
import jax.numpy as jnp
from jax import lax
from jax.experimental import pallas as pl
from jax.experimental.pallas import tpu as pltpu
from jax.experimental.pallas import tpu_sc as plsc

N = 10000
E = 320000
NREL = 3
IN_C = 128
OUT3 = 384
NCF = 8
F = 32
NSEG = 10240
SROWS = NREL * NSEG
DUMMY_DST = 10200
NC = 2
NS = 16
B = 128
NB = 80
EPT = NB * B
EPAD = NC * NS * EPT
RPT = SROWS // NS
CPT = NSEG // NS

_sc_mesh = plsc.VectorSubcoreMesh(core_axis_name="c", subcore_axis_name="s")


@functools.partial(
    pl.kernel,
    out_type=[
        jax.ShapeDtypeStruct((NC, NCF, SROWS, F), jnp.float32),
        jax.ShapeDtypeStruct((NC, NSEG, F), jnp.float32),
    ],
    mesh=_sc_mesh,
    compiler_params=pltpu.CompilerParams(use_tc_tiling_on_sc=False),
    scratch_types=[
        pltpu.VMEM((NB, B), jnp.int32),
        pltpu.VMEM((NB, B), jnp.int32),
        pltpu.VMEM((NB, B), jnp.int32),
        pltpu.VMEM((NB, B), jnp.int32),
        pltpu.VMEM((B, F), jnp.float32),
        pltpu.VMEM((B, F), jnp.float32),
        pltpu.VMEM((B, F), jnp.float32),
        pltpu.VMEM((B, F), jnp.float32),
        pltpu.VMEM((B, F), jnp.float32),
        pltpu.VMEM((B, F), jnp.float32),
        pltpu.SemaphoreType.DMA((4,)),
        pltpu.SemaphoreType.DMA((4,)),
        pltpu.SemaphoreType.DMA,
        pltpu.VMEM_SHARED((SROWS, F), jnp.float32),
    ],
)
def _sc_aggregate(xh_hbm, src_hbm, dst_hbm, et_hbm,
                  s_out, cnt_out,
                  src_v, sidx_v, dst_v, et_v, buf0, buf1, buf2, buf3,
                  zbuf, obuf, gsem, ssem, csem, S_sh):
    bufs = (buf0, buf1, buf2, buf3)
    c = lax.axis_index("c")
    s = lax.axis_index("s")
    tid = c * NS + s

    pltpu.sync_copy(src_hbm.at[tid], src_v)
    pltpu.sync_copy(dst_hbm.at[tid], dst_v)
    pltpu.sync_copy(et_hbm.at[tid], et_v)

    zero16f = jnp.zeros((16,), jnp.float32)
    ones16f = jnp.full((16,), 1.0, jnp.float32)

    nzc = F // 16

    def _fill_zo(t, _):
        zbuf[t // nzc, pl.ds((t % nzc) * 16, 16)] = zero16f
        obuf[t // nzc, pl.ds((t % nzc) * 16, 16)] = ones16f
        return 0
    lax.fori_loop(0, B * nzc, _fill_zo, 0)

    def _sidx_body(t, _):
        r = t // (B // 16)
        cb = (t % (B // 16)) * 16
        dv = dst_v[r, pl.ds(cb, 16)]
        ev = et_v[r, pl.ds(cb, 16)]
        sidx_v[r, pl.ds(cb, 16)] = ev * NSEG + dv
        return 0
    lax.fori_loop(0, NB * (B // 16), _sidx_body, 0)

    def _zero_all(u, _):
        pltpu.sync_copy(zbuf, S_sh.at[pl.ds(s * RPT + u * B, B)])
        return 0
    lax.fori_loop(0, RPT // B, _zero_all, 0)
    plsc.subcore_barrier()

    def _cnt_batch(j, _):
        pltpu.async_copy(obuf, S_sh.at[dst_v.at[j]], csem, add=True)
        return 0
    lax.fori_loop(0, NB, _cnt_batch, 0)

    def _cnt_drain(j, _):
        pltpu.make_async_copy(obuf, S_sh.at[dst_v.at[0]], csem).wait()
        return 0
    lax.fori_loop(0, NB, _cnt_drain, 0)
    plsc.subcore_barrier()
    for pc in range(NC):
        @pl.when(c == pc)
        def _():
            pltpu.sync_copy(S_sh.at[pl.ds(s * CPT, CPT)],
                            cnt_out.at[pc, pl.ds(s * CPT, CPT)])
    def _zero_cnt_rows(u, _):
        pltpu.sync_copy(zbuf, S_sh.at[pl.ds(s * CPT + u * B, B)])
        return 0
    lax.fori_loop(0, CPT // B, _zero_cnt_rows, 0)
    plsc.subcore_barrier()

    for k in range(NCF):
        if k > 0:
            def _zero_slab(u, _):
                pltpu.sync_copy(zbuf, S_sh.at[pl.ds(s * RPT + u * B, B)])
                return 0
            lax.fori_loop(0, RPT // B, _zero_slab, 0)
            plsc.subcore_barrier()

        xh_k = xh_hbm.at[k]

        def _g(j, b):
            pltpu.async_copy(xh_k.at[src_v.at[j]], bufs[b], gsem.at[b])

        def _gwait(b):
            pltpu.make_async_copy(xh_k.at[src_v.at[0]], bufs[b],
                                  gsem.at[b]).wait()

        def _s(j, b):
            pltpu.async_copy(bufs[b], S_sh.at[sidx_v.at[j]], ssem.at[b],
                             add=True)

        def _swait(b):
            pltpu.make_async_copy(bufs[b], S_sh.at[sidx_v.at[0]],
                                  ssem.at[b]).wait()

        _g(0, 0)
        _g(1, 1)

        def _ring(jj, _):
            for b in range(4):
                j = jj * 4 + b
                bn = (b + 2) % 4
                if b < 2:
                    @pl.when(jj > 0)
                    def _():
                        _swait(bn)
                    _g(j + 2, bn)
                else:
                    @pl.when(jj < NB // 4 - 1)
                    def _():
                        _swait(bn)
                        _g(j + 2, bn)
                _gwait(b)
                _s(j, b)
            return 0
        lax.fori_loop(0, NB // 4, _ring, 0)
        for b in range(4):
            _swait(b)
        plsc.subcore_barrier()

        for pc in range(NC):
            @pl.when(c == pc)
            def _():
                pltpu.sync_copy(S_sh.at[pl.ds(s * RPT, RPT)],
                                s_out.at[pc, k, pl.ds(s * RPT, RPT)])
        plsc.subcore_barrier()


BLK = 512
NBLK = NSEG // BLK


def _tc_body(s0_ref, s1_ref, s2_ref, cnt_ref, x_ref, h_ref,
             ai_ref, bi_ref, ri_ref, ah_ref, bh_ref, rh_ref,
             bii_ref, bir_ref, bin_ref, out_ref):
    Sx, Sh = [], []
    for ref in (s0_ref, s1_ref, s2_ref):
        Sr = ref[...]
        Spr = Sr[0] + Sr[1]
        Sx.append(jnp.concatenate([Spr[0], Spr[1], Spr[2], Spr[3]], axis=-1))
        Sh.append(jnp.concatenate([Spr[4], Spr[5], Spr[6], Spr[7]], axis=-1))
    cnt = cnt_ref[...][0]
    inv = (1.0 / jnp.maximum(cnt[0] + cnt[1], 1.0))[:, None]

    def conv(Sr, att_ref, basis_ref, xn, root_ref):
        agg = jnp.zeros((BLK, OUT3), jnp.float32)
        basis = basis_ref[...]
        for b in range(3):
            Yb = (att_ref[0, b] * Sr[0] + att_ref[1, b] * Sr[1]
                  + att_ref[2, b] * Sr[2])
            agg = agg + jnp.dot(Yb, basis[b],
                                preferred_element_type=jnp.float32)
        return agg * inv + jnp.dot(xn, root_ref[...],
                                   preferred_element_type=jnp.float32)

    x = x_ref[...]
    h = h_ref[...]
    gi = conv(Sx, ai_ref, bi_ref, x, ri_ref)
    gh = conv(Sh, ah_ref, bh_ref, h, rh_ref)
    rg = jax.nn.sigmoid(gi[:, :128] + gh[:, :128] + bir_ref[...])
    ig = jax.nn.sigmoid(gi[:, 128:256] + gh[:, 128:256] + bii_ref[...])
    ng = jnp.tanh(gi[:, 256:] + rg * gh[:, 256:] + bin_ref[...])
    out_ref[...] = (1.0 - ig) * ng + ig * h


def _s_spec(r):
    return pl.BlockSpec((NC, NCF, BLK, F), lambda i, r=r: (0, 0, r * NBLK + i, 0))


_tc_gates = pl.pallas_call(
    _tc_body,
    grid=(NBLK,),
    in_specs=[
        _s_spec(0), _s_spec(1), _s_spec(2),
        pl.BlockSpec((1, NC, BLK), lambda i: (i, 0, 0)),
        pl.BlockSpec((BLK, IN_C), lambda i: (i, 0)),
        pl.BlockSpec((BLK, IN_C), lambda i: (i, 0)),
        pl.BlockSpec(memory_space=pltpu.SMEM),
        pl.BlockSpec((NREL, IN_C, OUT3), lambda i: (0, 0, 0)),
        pl.BlockSpec((IN_C, OUT3), lambda i: (0, 0)),
        pl.BlockSpec(memory_space=pltpu.SMEM),
        pl.BlockSpec((NREL, IN_C, OUT3), lambda i: (0, 0, 0)),
        pl.BlockSpec((IN_C, OUT3), lambda i: (0, 0)),
        pl.BlockSpec((1, IN_C), lambda i: (0, 0)),
        pl.BlockSpec((1, IN_C), lambda i: (0, 0)),
        pl.BlockSpec((1, IN_C), lambda i: (0, 0)),
    ],
    out_specs=pl.BlockSpec((BLK, IN_C), lambda i: (i, 0)),
    out_shape=jax.ShapeDtypeStruct((NSEG, IN_C), jnp.float32),
)


def kernel(inputs, edge_index, edge_attr, hidden,
           basis_i, att_i, root_i, basis_h, att_h, root_h,
           bias_i, bias_r, bias_n):
    x = inputs
    h = hidden
    xh = jnp.concatenate([x, h], axis=1).reshape(N, NCF, F).transpose(1, 0, 2)
    src = edge_index[0]
    dst = edge_index[1]
    et = edge_attr
    pad = EPAD - E
    pidx = jnp.arange(pad, dtype=jnp.int32)
    srcp = jnp.concatenate(
        [src, pidx % jnp.int32(N)]).reshape(NC * NS, NB, B)
    dstp = jnp.concatenate(
        [dst, jnp.int32(N) + pidx % jnp.int32(NSEG - N)]).reshape(
            NC * NS, NB, B)
    etp = jnp.concatenate(
        [et, (pidx // jnp.int32(NSEG - N)) % jnp.int32(NREL)]).reshape(
            NC * NS, NB, B)

    s_p, cnt_p = _sc_aggregate(xh, srcp, dstp, etp)
    cnt2 = cnt_p[:, :, 0].reshape(NC, NBLK, BLK)
    cnt2 = cnt2.transpose(1, 0, 2)
    xpad = jnp.zeros((NSEG - N, IN_C), jnp.float32)
    xp = jnp.concatenate([x, xpad])
    hp = jnp.concatenate([h, xpad])

    out = _tc_gates(s_p, s_p, s_p, cnt2, xp, hp,
                    att_i, basis_i, root_i, att_h, basis_h, root_h,
                    bias_i.reshape(1, IN_C), bias_r.reshape(1, IN_C),
                    bias_n.reshape(1, IN_C))
    out = out[:N]
    return (out, out)


# speedup vs baseline: 1.3253x; 1.3253x over previous
import functools

import jax
import jax.numpy as jnp
from jax import lax
from jax.experimental import pallas as pl
from jax.experimental.pallas import tpu as pltpu
from jax.experimental.pallas import tpu_sc as plsc

N = 10000
E = 320000
NREL = 3
IN_C = 128
OUT3 = 384
NCF = 8
F = 32
NSEG = 10240
SROWS = NREL * NSEG
DUMMY_DST = 10200
NC = 2
NS = 16
B = 128
NB = 80
EPT = NB * B
EPAD = NC * NS * EPT
RPT = SROWS // NS
CPT = NSEG // NS

_sc_mesh = plsc.VectorSubcoreMesh(core_axis_name="c", subcore_axis_name="s")


@functools.partial(
    pl.kernel,
    out_type=[
        jax.ShapeDtypeStruct((NC, 2, SROWS, IN_C), jnp.float32),
        jax.ShapeDtypeStruct((NC, NSEG, F), jnp.float32),
    ],
    mesh=_sc_mesh,
    compiler_params=pltpu.CompilerParams(use_tc_tiling_on_sc=False),
    scratch_types=[
        pltpu.VMEM((NB, B), jnp.int32),
        pltpu.VMEM((NB, B), jnp.int32),
        pltpu.VMEM((NB, B), jnp.int32),
        pltpu.VMEM((NB, B), jnp.int32),
        pltpu.VMEM((B, F), jnp.float32),
        pltpu.VMEM((B, F), jnp.float32),
        pltpu.VMEM((B, F), jnp.float32),
        pltpu.VMEM((B, F), jnp.float32),
        pltpu.VMEM((B, F), jnp.float32),
        pltpu.VMEM((B, F), jnp.float32),
        pltpu.SemaphoreType.DMA((4,)),
        pltpu.SemaphoreType.DMA((4,)),
        pltpu.SemaphoreType.DMA,
        pltpu.VMEM_SHARED((SROWS, F), jnp.float32),
    ],
)
def _sc_aggregate(xh_hbm, src_hbm, dst_hbm, et_hbm,
                  s_out, cnt_out,
                  src_v, sidx_v, dst_v, et_v, buf0, buf1, buf2, buf3,
                  zbuf, obuf, gsem, ssem, csem, S_sh):
    bufs = (buf0, buf1, buf2, buf3)
    c = lax.axis_index("c")
    s = lax.axis_index("s")
    tid = c * NS + s

    pltpu.sync_copy(src_hbm.at[tid], src_v)
    pltpu.sync_copy(dst_hbm.at[tid], dst_v)
    pltpu.sync_copy(et_hbm.at[tid], et_v)

    zero16f = jnp.zeros((16,), jnp.float32)
    ones16f = jnp.full((16,), 1.0, jnp.float32)

    nzc = F // 16

    def _fill_zo(t, _):
        zbuf[t // nzc, pl.ds((t % nzc) * 16, 16)] = zero16f
        obuf[t // nzc, pl.ds((t % nzc) * 16, 16)] = ones16f
        return 0
    lax.fori_loop(0, B * nzc, _fill_zo, 0)

    def _sidx_body(t, _):
        r = t // (B // 16)
        cb = (t % (B // 16)) * 16
        dv = dst_v[r, pl.ds(cb, 16)]
        ev = et_v[r, pl.ds(cb, 16)]
        sidx_v[r, pl.ds(cb, 16)] = ev * NSEG + dv
        return 0
    lax.fori_loop(0, NB * (B // 16), _sidx_body, 0)

    def _zero_all(u, _):
        pltpu.sync_copy(zbuf, S_sh.at[pl.ds(s * RPT + u * B, B)])
        return 0
    lax.fori_loop(0, RPT // B, _zero_all, 0)
    plsc.subcore_barrier()

    def _cnt_batch(j, _):
        pltpu.async_copy(obuf, S_sh.at[dst_v.at[j]], csem, add=True)
        return 0
    lax.fori_loop(0, NB, _cnt_batch, 0)

    def _cnt_drain(j, _):
        pltpu.make_async_copy(obuf, S_sh.at[dst_v.at[0]], csem).wait()
        return 0
    lax.fori_loop(0, NB, _cnt_drain, 0)
    plsc.subcore_barrier()
    for pc in range(NC):
        @pl.when(c == pc)
        def _():
            pltpu.sync_copy(S_sh.at[pl.ds(s * CPT, CPT)],
                            cnt_out.at[pc, pl.ds(s * CPT, CPT)])
    def _zero_cnt_rows(u, _):
        pltpu.sync_copy(zbuf, S_sh.at[pl.ds(s * CPT + u * B, B)])
        return 0
    lax.fori_loop(0, CPT // B, _zero_cnt_rows, 0)
    plsc.subcore_barrier()

    for k in range(NCF):
        if k > 0:
            def _zero_slab(u, _):
                pltpu.sync_copy(zbuf, S_sh.at[pl.ds(s * RPT + u * B, B)])
                return 0
            lax.fori_loop(0, RPT // B, _zero_slab, 0)
            plsc.subcore_barrier()

        xh_k = xh_hbm.at[k]

        def _g(j, b):
            pltpu.async_copy(xh_k.at[src_v.at[j]], bufs[b], gsem.at[b])

        def _gwait(b):
            pltpu.make_async_copy(xh_k.at[src_v.at[0]], bufs[b],
                                  gsem.at[b]).wait()

        def _s(j, b):
            pltpu.async_copy(bufs[b], S_sh.at[sidx_v.at[j]], ssem.at[b],
                             add=True)

        def _swait(b):
            pltpu.make_async_copy(bufs[b], S_sh.at[sidx_v.at[0]],
                                  ssem.at[b]).wait()

        _g(0, 0)
        _g(1, 1)

        def _ring(jj, _):
            for b in range(4):
                j = jj * 4 + b
                bn = (b + 2) % 4
                if b < 2:
                    @pl.when(jj > 0)
                    def _():
                        _swait(bn)
                    _g(j + 2, bn)
                else:
                    @pl.when(jj < NB // 4 - 1)
                    def _():
                        _swait(bn)
                        _g(j + 2, bn)
                _gwait(b)
                _s(j, b)
            return 0
        lax.fori_loop(0, NB // 4, _ring, 0)
        for b in range(4):
            _swait(b)
        plsc.subcore_barrier()

        for pc in range(NC):
            @pl.when(c == pc)
            def _():
                pltpu.sync_copy(
                    S_sh.at[pl.ds(s * RPT, RPT)],
                    s_out.at[pc, k // 4, pl.ds(s * RPT, RPT),
                             pl.ds((k % 4) * F, F)])
        plsc.subcore_barrier()


BLK = 512
NBLK = NSEG // BLK


def _tc_body(s0_ref, s1_ref, s2_ref, cnt_ref, x_ref, h_ref,
             ai_ref, bi_ref, ri_ref, ah_ref, bh_ref, rh_ref,
             bii_ref, bir_ref, bin_ref, out_ref):
    Sx, Sh = [], []
    for ref in (s0_ref, s1_ref, s2_ref):
        Sr = ref[...]
        Spr = Sr[0] + Sr[1]
        Sx.append(Spr[0])
        Sh.append(Spr[1])
    cnt = cnt_ref[...][0]
    inv = (1.0 / jnp.maximum(cnt[0] + cnt[1], 1.0))[:, None]

    def conv(Sr, att_ref, basis_ref, xn, root_ref):
        agg = jnp.zeros((BLK, OUT3), jnp.float32)
        basis = basis_ref[...]
        for b in range(3):
            Yb = (att_ref[0, b] * Sr[0] + att_ref[1, b] * Sr[1]
                  + att_ref[2, b] * Sr[2])
            agg = agg + jnp.dot(Yb, basis[b],
                                preferred_element_type=jnp.float32)
        return agg * inv + jnp.dot(xn, root_ref[...],
                                   preferred_element_type=jnp.float32)

    x = x_ref[...]
    h = h_ref[...]
    gi = conv(Sx, ai_ref, bi_ref, x, ri_ref)
    gh = conv(Sh, ah_ref, bh_ref, h, rh_ref)
    rg = jax.nn.sigmoid(gi[:, :128] + gh[:, :128] + bir_ref[...])
    ig = jax.nn.sigmoid(gi[:, 128:256] + gh[:, 128:256] + bii_ref[...])
    ng = jnp.tanh(gi[:, 256:] + rg * gh[:, 256:] + bin_ref[...])
    out_ref[...] = (1.0 - ig) * ng + ig * h


def _s_spec(r):
    return pl.BlockSpec((NC, 2, BLK, IN_C),
                        lambda i, r=r: (0, 0, r * NBLK + i, 0))


_tc_gates = pl.pallas_call(
    _tc_body,
    grid=(NBLK,),
    in_specs=[
        _s_spec(0), _s_spec(1), _s_spec(2),
        pl.BlockSpec((1, NC, BLK), lambda i: (i, 0, 0)),
        pl.BlockSpec((BLK, IN_C), lambda i: (i, 0)),
        pl.BlockSpec((BLK, IN_C), lambda i: (i, 0)),
        pl.BlockSpec(memory_space=pltpu.SMEM),
        pl.BlockSpec((NREL, IN_C, OUT3), lambda i: (0, 0, 0)),
        pl.BlockSpec((IN_C, OUT3), lambda i: (0, 0)),
        pl.BlockSpec(memory_space=pltpu.SMEM),
        pl.BlockSpec((NREL, IN_C, OUT3), lambda i: (0, 0, 0)),
        pl.BlockSpec((IN_C, OUT3), lambda i: (0, 0)),
        pl.BlockSpec((1, IN_C), lambda i: (0, 0)),
        pl.BlockSpec((1, IN_C), lambda i: (0, 0)),
        pl.BlockSpec((1, IN_C), lambda i: (0, 0)),
    ],
    out_specs=pl.BlockSpec((BLK, IN_C), lambda i: (i, 0)),
    out_shape=jax.ShapeDtypeStruct((NSEG, IN_C), jnp.float32),
)


def kernel(inputs, edge_index, edge_attr, hidden,
           basis_i, att_i, root_i, basis_h, att_h, root_h,
           bias_i, bias_r, bias_n):
    x = inputs
    h = hidden
    xh = jnp.concatenate([x, h], axis=1).reshape(N, NCF, F).transpose(1, 0, 2)
    src = edge_index[0]
    dst = edge_index[1]
    et = edge_attr
    pad = EPAD - E
    pidx = jnp.arange(pad, dtype=jnp.int32)
    srcp = jnp.concatenate(
        [src, pidx % jnp.int32(N)]).reshape(NC * NS, NB, B)
    dstp = jnp.concatenate(
        [dst, jnp.int32(N) + pidx % jnp.int32(NSEG - N)]).reshape(
            NC * NS, NB, B)
    etp = jnp.concatenate(
        [et, (pidx // jnp.int32(NSEG - N)) % jnp.int32(NREL)]).reshape(
            NC * NS, NB, B)

    s_p, cnt_p = _sc_aggregate(xh, srcp, dstp, etp)
    cnt2 = cnt_p[:, :, 0].reshape(NC, NBLK, BLK)
    cnt2 = cnt2.transpose(1, 0, 2)
    xpad = jnp.zeros((NSEG - N, IN_C), jnp.float32)
    xp = jnp.concatenate([x, xpad])
    hp = jnp.concatenate([h, xpad])

    out = _tc_gates(s_p, s_p, s_p, cnt2, xp, hp,
                    att_i, basis_i, root_i, att_h, basis_h, root_h,
                    bias_i.reshape(1, IN_C), bias_r.reshape(1, IN_C),
                    bias_n.reshape(1, IN_C))
    out = out[:N]
    return (out, out)


# speedup vs baseline: 1.3384x; 1.0099x over previous
import functools

import jax
import jax.numpy as jnp
from jax import lax
from jax.experimental import pallas as pl
from jax.experimental.pallas import tpu as pltpu
from jax.experimental.pallas import tpu_sc as plsc

N = 10000
E = 320000
NREL = 3
IN_C = 128
OUT3 = 384
NCF = 8
F = 32
NSEG = 10240
SROWS = NREL * NSEG
DUMMY_DST = 10200
NC = 2
NS = 16
B = 128
NB = 80
EPT = NB * B
EPAD = NC * NS * EPT
RPT = SROWS // NS
CPT = NSEG // NS

_sc_mesh = plsc.VectorSubcoreMesh(core_axis_name="c", subcore_axis_name="s")


@functools.partial(
    pl.kernel,
    out_type=[
        jax.ShapeDtypeStruct((NC, 2, SROWS, IN_C), jnp.float32),
        jax.ShapeDtypeStruct((NC, NSEG, F), jnp.float32),
    ],
    mesh=_sc_mesh,
    compiler_params=pltpu.CompilerParams(use_tc_tiling_on_sc=False),
    scratch_types=[
        pltpu.VMEM((NB, B), jnp.int32),
        pltpu.VMEM((NB, B), jnp.int32),
        pltpu.VMEM((NB, B), jnp.int32),
        pltpu.VMEM((NB, B), jnp.int32),
        pltpu.VMEM((B, F), jnp.float32),
        pltpu.VMEM((B, F), jnp.float32),
        pltpu.VMEM((B, F), jnp.float32),
        pltpu.VMEM((B, F), jnp.float32),
        pltpu.VMEM((B, F), jnp.float32),
        pltpu.VMEM((B, F), jnp.float32),
        pltpu.VMEM((B, F), jnp.float32),
        pltpu.VMEM((B, F), jnp.float32),
        pltpu.SemaphoreType.DMA((6,)),
        pltpu.SemaphoreType.DMA((6,)),
        pltpu.SemaphoreType.DMA,
        pltpu.VMEM_SHARED((SROWS, F), jnp.float32),
    ],
)
def _sc_aggregate(xh_hbm, src_hbm, dst_hbm, et_hbm,
                  s_out, cnt_out,
                  src_v, sidx_v, dst_v, et_v, buf0, buf1, buf2, buf3,
                  buf4, buf5, zbuf, obuf, gsem, ssem, csem, S_sh):
    bufs = (buf0, buf1, buf2, buf3, buf4, buf5)
    c = lax.axis_index("c")
    s = lax.axis_index("s")
    tid = c * NS + s

    pltpu.sync_copy(src_hbm.at[tid], src_v)
    pltpu.sync_copy(dst_hbm.at[tid], dst_v)
    pltpu.sync_copy(et_hbm.at[tid], et_v)

    zero16f = jnp.zeros((16,), jnp.float32)
    ones16f = jnp.full((16,), 1.0, jnp.float32)

    nzc = F // 16

    def _fill_zo(t, _):
        zbuf[t // nzc, pl.ds((t % nzc) * 16, 16)] = zero16f
        obuf[t // nzc, pl.ds((t % nzc) * 16, 16)] = ones16f
        return 0
    lax.fori_loop(0, B * nzc, _fill_zo, 0)

    def _sidx_body(t, _):
        r = t // (B // 16)
        cb = (t % (B // 16)) * 16
        dv = dst_v[r, pl.ds(cb, 16)]
        ev = et_v[r, pl.ds(cb, 16)]
        sidx_v[r, pl.ds(cb, 16)] = ev * NSEG + dv
        return 0
    lax.fori_loop(0, NB * (B // 16), _sidx_body, 0)

    def _zero_all(u, _):
        pltpu.sync_copy(zbuf, S_sh.at[pl.ds(s * RPT + u * B, B)])
        return 0
    lax.fori_loop(0, RPT // B, _zero_all, 0)
    plsc.subcore_barrier()

    def _cnt_batch(j, _):
        pltpu.async_copy(obuf, S_sh.at[dst_v.at[j]], csem, add=True)
        return 0
    lax.fori_loop(0, NB, _cnt_batch, 0)

    def _cnt_drain(j, _):
        pltpu.make_async_copy(obuf, S_sh.at[dst_v.at[0]], csem).wait()
        return 0
    lax.fori_loop(0, NB, _cnt_drain, 0)
    plsc.subcore_barrier()
    for pc in range(NC):
        @pl.when(c == pc)
        def _():
            pltpu.sync_copy(S_sh.at[pl.ds(s * CPT, CPT)],
                            cnt_out.at[pc, pl.ds(s * CPT, CPT)])
    def _zero_cnt_rows(u, _):
        pltpu.sync_copy(zbuf, S_sh.at[pl.ds(s * CPT + u * B, B)])
        return 0
    lax.fori_loop(0, CPT // B, _zero_cnt_rows, 0)
    plsc.subcore_barrier()

    for k in range(NCF):
        if k > 0:
            def _zero_slab(u, _):
                pltpu.sync_copy(zbuf, S_sh.at[pl.ds(s * RPT + u * B, B)])
                return 0
            lax.fori_loop(0, RPT // B, _zero_slab, 0)
            plsc.subcore_barrier()

        xh_k = xh_hbm.at[k]

        def _g(j, b):
            pltpu.async_copy(xh_k.at[src_v.at[j]], bufs[b], gsem.at[b])

        def _gwait(b):
            pltpu.make_async_copy(xh_k.at[src_v.at[0]], bufs[b],
                                  gsem.at[b]).wait()

        def _s(j, b):
            pltpu.async_copy(bufs[b], S_sh.at[sidx_v.at[j]], ssem.at[b],
                             add=True)

        def _swait(b):
            pltpu.make_async_copy(bufs[b], S_sh.at[sidx_v.at[0]],
                                  ssem.at[b]).wait()

        _g(0, 0)
        _g(1, 1)

        def _ring(jj, _):
            for b in range(4):
                j = jj * 4 + b
                bn = (b + 2) % 4
                if b < 2:
                    @pl.when(jj > 0)
                    def _():
                        _swait(bn)
                    _g(j + 2, bn)
                else:
                    @pl.when(jj < NB // 4 - 1)
                    def _():
                        _swait(bn)
                        _g(j + 2, bn)
                _gwait(b)
                _s(j, b)
            return 0
        lax.fori_loop(0, NB // 4, _ring, 0)
        for b in range(4):
            _swait(b)
        plsc.subcore_barrier()

        for pc in range(NC):
            @pl.when(c == pc)
            def _():
                pltpu.sync_copy(
                    S_sh.at[pl.ds(s * RPT, RPT)],
                    s_out.at[pc, k // 4, pl.ds(s * RPT, RPT),
                             pl.ds((k % 4) * F, F)])
        plsc.subcore_barrier()


BLK = 512
NBLK = NSEG // BLK


def _tc_body(s0_ref, s1_ref, s2_ref, cnt_ref, x_ref, h_ref,
             ai_ref, bi_ref, ri_ref, ah_ref, bh_ref, rh_ref,
             bii_ref, bir_ref, bin_ref, out_ref):
    Sx, Sh = [], []
    for ref in (s0_ref, s1_ref, s2_ref):
        Sr = ref[...]
        Spr = Sr[0] + Sr[1]
        Sx.append(Spr[0])
        Sh.append(Spr[1])
    cnt = cnt_ref[...][0]
    inv = (1.0 / jnp.maximum(cnt[0] + cnt[1], 1.0))[:, None]

    def conv(Sr, att_ref, basis_ref, xn, root_ref):
        agg = jnp.zeros((BLK, OUT3), jnp.float32)
        basis = basis_ref[...]
        for b in range(3):
            Yb = (att_ref[0, b] * Sr[0] + att_ref[1, b] * Sr[1]
                  + att_ref[2, b] * Sr[2])
            agg = agg + jnp.dot(Yb, basis[b],
                                preferred_element_type=jnp.float32)
        return agg * inv + jnp.dot(xn, root_ref[...],
                                   preferred_element_type=jnp.float32)

    x = x_ref[...]
    h = h_ref[...]
    gi = conv(Sx, ai_ref, bi_ref, x, ri_ref)
    gh = conv(Sh, ah_ref, bh_ref, h, rh_ref)
    rg = jax.nn.sigmoid(gi[:, :128] + gh[:, :128] + bir_ref[...])
    ig = jax.nn.sigmoid(gi[:, 128:256] + gh[:, 128:256] + bii_ref[...])
    ng = jnp.tanh(gi[:, 256:] + rg * gh[:, 256:] + bin_ref[...])
    out_ref[...] = (1.0 - ig) * ng + ig * h


def _s_spec(r):
    return pl.BlockSpec((NC, 2, BLK, IN_C),
                        lambda i, r=r: (0, 0, r * NBLK + i, 0))


_tc_gates = pl.pallas_call(
    _tc_body,
    grid=(NBLK,),
    in_specs=[
        _s_spec(0), _s_spec(1), _s_spec(2),
        pl.BlockSpec((1, NC, BLK), lambda i: (i, 0, 0)),
        pl.BlockSpec((BLK, IN_C), lambda i: (i, 0)),
        pl.BlockSpec((BLK, IN_C), lambda i: (i, 0)),
        pl.BlockSpec(memory_space=pltpu.SMEM),
        pl.BlockSpec((NREL, IN_C, OUT3), lambda i: (0, 0, 0)),
        pl.BlockSpec((IN_C, OUT3), lambda i: (0, 0)),
        pl.BlockSpec(memory_space=pltpu.SMEM),
        pl.BlockSpec((NREL, IN_C, OUT3), lambda i: (0, 0, 0)),
        pl.BlockSpec((IN_C, OUT3), lambda i: (0, 0)),
        pl.BlockSpec((1, IN_C), lambda i: (0, 0)),
        pl.BlockSpec((1, IN_C), lambda i: (0, 0)),
        pl.BlockSpec((1, IN_C), lambda i: (0, 0)),
    ],
    out_specs=pl.BlockSpec((BLK, IN_C), lambda i: (i, 0)),
    out_shape=jax.ShapeDtypeStruct((NSEG, IN_C), jnp.float32),
)


def kernel(inputs, edge_index, edge_attr, hidden,
           basis_i, att_i, root_i, basis_h, att_h, root_h,
           bias_i, bias_r, bias_n):
    x = inputs
    h = hidden
    xh = jnp.concatenate([x, h], axis=1).reshape(N, NCF, F).transpose(1, 0, 2)
    src = edge_index[0]
    dst = edge_index[1]
    et = edge_attr
    pad = EPAD - E
    pidx = jnp.arange(pad, dtype=jnp.int32)
    srcp = jnp.concatenate(
        [src, pidx % jnp.int32(N)]).reshape(NC * NS, NB, B)
    dstp = jnp.concatenate(
        [dst, jnp.int32(N) + pidx % jnp.int32(NSEG - N)]).reshape(
            NC * NS, NB, B)
    etp = jnp.concatenate(
        [et, (pidx // jnp.int32(NSEG - N)) % jnp.int32(NREL)]).reshape(
            NC * NS, NB, B)

    s_p, cnt_p = _sc_aggregate(xh, srcp, dstp, etp)
    cnt2 = cnt_p[:, :, 0].reshape(NC, NBLK, BLK)
    cnt2 = cnt2.transpose(1, 0, 2)
    xpad = jnp.zeros((NSEG - N, IN_C), jnp.float32)
    xp = jnp.concatenate([x, xpad])
    hp = jnp.concatenate([h, xpad])

    out = _tc_gates(s_p, s_p, s_p, cnt2, xp, hp,
                    att_i, basis_i, root_i, att_h, basis_h, root_h,
                    bias_i.reshape(1, IN_C), bias_r.reshape(1, IN_C),
                    bias_n.reshape(1, IN_C))
    out = out[:N]
    return (out, out)
